# Initial kernel scaffold; baseline (speedup 1.0000x reference)
import jax
import jax.numpy as jnp
from jax.experimental import pallas as pl


def kernel(imgs, batch_points):
    raise NotImplementedError("write your pallas kernel here")


# speedup vs baseline: 3.8906x; 3.8906x over previous
import functools

import jax
import jax.numpy as jnp
from jax import lax
from jax.experimental import pallas as pl
from jax.experimental.pallas import tpu as pltpu
from jax.experimental.pallas import tpu_sc as plsc

_B, _C, _H, _W = 4, 192, 224, 224
_P = 64
_HW = _H * _W
_CHW = _C * _HW
_NW = 32
_ROWS = _B * _P
_RPW = _ROWS // _NW
_LANES = 16
_CPR = _C // _LANES
_NT = _RPW * _C // 128


@jax.jit
def _crop_call(imgs_flat, pts_flat):
    mesh = plsc.VectorSubcoreMesh(core_axis_name="c", subcore_axis_name="s")

    @functools.partial(
        pl.kernel,
        mesh=mesh,
        out_type=jax.ShapeDtypeStruct((_ROWS * _C,), jnp.float32),
        scratch_types=[
            pltpu.VMEM((_LANES,), jnp.int32),
            pltpu.VMEM((_NT, 128), jnp.int32),
            pltpu.VMEM((_NT * 128,), jnp.float32),
            pltpu.SemaphoreType.DMA,
        ],
        compiler_params=pltpu.CompilerParams(needs_layout_passes=False),
    )
    def crop(imgs_hbm, pts_hbm, out_hbm, pts_v, idx_v, crop_v, sem):
        wid = lax.axis_index("s") * 2 + lax.axis_index("c")
        pltpu.sync_copy(pts_hbm.at[pl.ds(wid * 16, 16)], pts_v)
        base = (wid // 8) * _CHW
        lanes = lax.iota(jnp.int32, 16)
        pv = pts_v[...]
        zero = jnp.zeros((16,), jnp.int32)
        for j in range(_RPW):
            xj = jnp.sum(jnp.where(lanes == 2 * j, pv, zero))
            yj = jnp.sum(jnp.where(lanes == 2 * j + 1, pv, zero))
            pix = base + yj * _W + xj
            for ci in range(_CPR):
                k = j * _CPR + ci
                idx = pix + (ci * 16 + lanes) * _HW
                idx_v[k // 8, pl.ds((k % 8) * 16, 16)] = idx
        copies = [
            pltpu.async_copy(
                imgs_hbm.at[idx_v.at[t]], crop_v.at[pl.ds(t * 128, 128)], sem
            )
            for t in range(_NT)
        ]
        for c in copies:
            c.wait()
        pltpu.sync_copy(crop_v, out_hbm.at[pl.ds(wid * _NT * 128, _NT * 128)])

    return crop(imgs_flat, pts_flat)


def kernel(imgs, batch_points):
    imgs_flat = imgs.reshape(-1)
    pts_flat = batch_points.astype(jnp.int32).reshape(-1)
    out = _crop_call(imgs_flat, pts_flat)
    return out.reshape(_ROWS, _C), _ROWS


# speedup vs baseline: 4.7598x; 1.2234x over previous
import functools

import jax
import jax.numpy as jnp
from jax import lax
from jax.experimental import pallas as pl
from jax.experimental.pallas import tpu as pltpu
from jax.experimental.pallas import tpu_sc as plsc

_B, _C, _H, _W = 4, 192, 224, 224
_P = 64
_PLANE = _H * 128
_NW = 32
_ROWS = _B * _P
_RPW = _ROWS // _NW
_LANES = 16
_CPR = _C // _LANES
_NT = _RPW * _C // 128


@jax.jit
def _crop_call(lo_flat, hi_flat, pts_flat):
    mesh = plsc.VectorSubcoreMesh(core_axis_name="c", subcore_axis_name="s")

    @functools.partial(
        pl.kernel,
        mesh=mesh,
        out_type=jax.ShapeDtypeStruct((_ROWS * _C,), jnp.float32),
        scratch_types=[
            pltpu.VMEM((_LANES,), jnp.int32),
            pltpu.VMEM((_NT, 128), jnp.int32),
            pltpu.VMEM((_NT, 128), jnp.int32),
            pltpu.VMEM((_NT * 128,), jnp.float32),
            pltpu.VMEM((_NT * 128,), jnp.float32),
            pltpu.VMEM((_NT * 128,), jnp.float32),
            pltpu.SemaphoreType.DMA,
        ],
        compiler_params=pltpu.CompilerParams(needs_layout_passes=False),
    )
    def crop(lo_hbm, hi_hbm, pts_hbm, out_hbm, pts_v, ilo_v, ihi_v,
             clo_v, chi_v, out_v, sem):
        wid = lax.axis_index("s") * 2 + lax.axis_index("c")
        pltpu.sync_copy(pts_hbm.at[pl.ds(wid * 16, 16)], pts_v)
        base = (wid // 8) * _C
        lanes = lax.iota(jnp.int32, 16)
        pv = pts_v[...]
        zero = jnp.zeros((16,), jnp.int32)
        sels = []
        for j in range(_RPW):
            xj = jnp.sum(jnp.where(lanes == 2 * j, pv, zero))
            yj = jnp.sum(jnp.where(lanes == 2 * j + 1, pv, zero))
            sels.append(xj < 128)
            pix = yj * 128
            plo = pix + jnp.minimum(xj, 127)
            phi = pix + jnp.maximum(xj - 128, 0)
            for ci in range(_CPR):
                k = j * _CPR + ci
                poff = (base + ci * 16 + lanes) * _PLANE
                dst = (k // 8, pl.ds((k % 8) * 16, 16))
                ilo_v[dst[0], dst[1]] = poff + plo
                ihi_v[dst[0], dst[1]] = poff + phi
        copies = [
            pltpu.async_copy(
                src.at[idx.at[t]], dst.at[pl.ds(t * 128, 128)], sem
            )
            for src, idx, dst in ((lo_hbm, ilo_v, clo_v), (hi_hbm, ihi_v, chi_v))
            for t in range(_NT)
        ]
        for c in copies:
            c.wait()
        for j in range(_RPW):
            sel = jnp.full((16,), sels[j], jnp.bool_)
            for ci in range(_CPR):
                o = j * _C + ci * 16
                out_v[pl.ds(o, 16)] = jnp.where(
                    sel, clo_v[pl.ds(o, 16)], chi_v[pl.ds(o, 16)]
                )
        pltpu.sync_copy(out_v, out_hbm.at[pl.ds(wid * _NT * 128, _NT * 128)])

    return crop(lo_flat, hi_flat, pts_flat)


def kernel(imgs, batch_points):
    lo_flat = imgs[:, :, :, :128].reshape(-1)
    hi_flat = jnp.pad(
        imgs[:, :, :, 128:], ((0, 0), (0, 0), (0, 0), (0, 32))
    ).reshape(-1)
    pts_flat = batch_points.astype(jnp.int32).reshape(-1)
    out = _crop_call(lo_flat, hi_flat, pts_flat)
    return out.reshape(_ROWS, _C), _ROWS


# speedup vs baseline: 6.1587x; 1.2939x over previous
import functools

import jax
import jax.numpy as jnp
from jax import lax
from jax.experimental import pallas as pl
from jax.experimental.pallas import tpu as pltpu
from jax.experimental.pallas import tpu_sc as plsc

_B, _C, _H, _W = 4, 192, 224, 224
_P = 64
_PLANE = _H * 128
_NW = 32
_ROWS = _B * _P
_RPW = _ROWS // _NW
_LANES = 16
_CPR = _C // _LANES
_HALF = _C // 2
_NT = _RPW * _C // 128


@jax.jit
def _crop_call(imgs_rows, hi_flat, pts_flat):
    mesh = plsc.VectorSubcoreMesh(core_axis_name="c", subcore_axis_name="s")

    @functools.partial(
        pl.kernel,
        mesh=mesh,
        out_type=jax.ShapeDtypeStruct((_ROWS * _C,), jnp.float32),
        scratch_types=[
            pltpu.VMEM((_LANES,), jnp.int32),
            pltpu.VMEM((2 * _RPW, _HALF), jnp.int32),
            pltpu.VMEM((_NT, 128), jnp.int32),
            pltpu.VMEM((2, _C, 128), jnp.float32),
            pltpu.VMEM((_NT * 128,), jnp.float32),
            pltpu.VMEM((_NT * 128,), jnp.float32),
            pltpu.SemaphoreType.DMA,
            pltpu.SemaphoreType.DMA,
            pltpu.SemaphoreType.DMA,
        ],
        compiler_params=pltpu.CompilerParams(needs_layout_passes=False),
    )
    def crop(imgs_hbm, hi_hbm, pts_hbm, out_hbm, pts_v, ridx_v, ihi_v,
             stage_v, chi_v, out_v, sem0, sem1, semh):
        lo = imgs_hbm.at[:, pl.ds(0, 128)]
        sems = (sem0, sem1)
        wid = lax.axis_index("s") * 2 + lax.axis_index("c")
        pltpu.sync_copy(pts_hbm.at[pl.ds(wid * 16, 16)], pts_v)
        base = (wid // 8) * _C
        lanes = lax.iota(jnp.int32, 16)
        pv = pts_v[...]
        zero = jnp.zeros((16,), jnp.int32)
        sels, xlos = [], []
        for j in range(_RPW):
            xj = jnp.sum(jnp.where(lanes == 2 * j, pv, zero))
            yj = jnp.sum(jnp.where(lanes == 2 * j + 1, pv, zero))
            sels.append(xj < 128)
            xlos.append(jnp.minimum(xj, 127))
            phi = yj * 128 + jnp.maximum(xj - 128, 0)
            for h in range(2):
                for k in range(_HALF // 16):
                    c = h * _HALF + k * 16 + lanes
                    ridx_v[2 * j + h, pl.ds(k * 16, 16)] = (
                        (base + c) * _H + yj
                    )
            for ci in range(_CPR):
                k = j * _CPR + ci
                ihi_v[k // 8, pl.ds((k % 8) * 16, 16)] = (
                    (base + ci * 16 + lanes) * _PLANE + phi
                )
        hi_copies = [
            pltpu.async_copy(
                hi_hbm.at[ihi_v.at[t]], chi_v.at[pl.ds(t * 128, 128)], semh
            )
            for t in range(_NT)
        ]

        def fire(j, buf):
            return [
                pltpu.async_copy(
                    lo.at[ridx_v.at[2 * j + h]],
                    stage_v.at[buf, pl.ds(h * _HALF, _HALF)],
                    sems[buf],
                )
                for h in range(2)
            ]

        pending = fire(0, 0)
        for j in range(_RPW):
            nxt = fire(j + 1, (j + 1) % 2) if j + 1 < _RPW else None
            for cpy in pending:
                cpy.wait()
            buf = j % 2
            xvec = jnp.full((16,), xlos[j], jnp.int32)
            for ci in range(_CPR):
                vals = plsc.load_gather(
                    stage_v.at[buf], [ci * 16 + lanes, xvec]
                )
                out_v[pl.ds(j * _C + ci * 16, 16)] = vals
            pending = nxt
        for cpy in hi_copies:
            cpy.wait()
        for j in range(_RPW):
            sel = jnp.full((16,), sels[j], jnp.bool_)
            for ci in range(_CPR):
                o = j * _C + ci * 16
                out_v[pl.ds(o, 16)] = jnp.where(
                    sel, out_v[pl.ds(o, 16)], chi_v[pl.ds(o, 16)]
                )
        pltpu.sync_copy(out_v, out_hbm.at[pl.ds(wid * _NT * 128, _NT * 128)])

    return crop(imgs_rows, hi_flat, pts_flat)


def kernel(imgs, batch_points):
    imgs_rows = imgs.reshape(_B * _C * _H, _W)
    hi_flat = jnp.pad(
        imgs[:, :, :, 128:], ((0, 0), (0, 0), (0, 0), (0, 32))
    ).reshape(-1)
    pts_flat = batch_points.astype(jnp.int32).reshape(-1)
    out = _crop_call(imgs_rows, hi_flat, pts_flat)
    return out.reshape(_ROWS, _C), _ROWS


# speedup vs baseline: 7.5940x; 1.2330x over previous
import functools

import jax
import jax.numpy as jnp
from jax import lax
from jax.experimental import pallas as pl
from jax.experimental.pallas import tpu as pltpu
from jax.experimental.pallas import tpu_sc as plsc

_B, _C, _H, _W = 4, 192, 224, 224
_P = 64
_PLANE = _H * 128
_NW = 32
_ROWS = _B * _P
_RPW = _ROWS // _NW
_LANES = 16
_CPR = _C // _LANES
_HALF = _C // 2
_NT = _RPW * _C // 128


@jax.jit
def _crop_call(imgs_rows, hi_flat, pts_flat):
    mesh = plsc.VectorSubcoreMesh(core_axis_name="c", subcore_axis_name="s")

    @functools.partial(
        pl.kernel,
        mesh=mesh,
        out_type=jax.ShapeDtypeStruct((_ROWS * _C,), jnp.float32),
        scratch_types=[
            pltpu.VMEM((_LANES,), jnp.int32),
            pltpu.VMEM((2 * _RPW, _HALF), jnp.int32),
            pltpu.VMEM((_NT, 128), jnp.int32),
            pltpu.VMEM((2, _C, 128), jnp.float32),
            pltpu.VMEM((_NT * 128,), jnp.float32),
            pltpu.VMEM((_NT * 128,), jnp.float32),
            pltpu.SemaphoreType.DMA,
            pltpu.SemaphoreType.DMA,
            pltpu.SemaphoreType.DMA,
        ],
        compiler_params=pltpu.CompilerParams(needs_layout_passes=False),
    )
    def crop(imgs_hbm, hi_hbm, pts_hbm, out_hbm, pts_v, ridx_v, ihi_v,
             stage_v, chi_v, out_v, sem0, sem1, semh):
        lo = imgs_hbm.at[:, pl.ds(0, 128)]
        sems = (sem0, sem1)
        wid = lax.axis_index("s") * 2 + lax.axis_index("c")
        pltpu.sync_copy(pts_hbm.at[pl.ds(wid * 16, 16)], pts_v)
        base = (wid // 8) * _C
        lanes = lax.iota(jnp.int32, 16)
        pv = pts_v[...]
        zero = jnp.zeros((16,), jnp.int32)
        sels, xlos = [], []
        for j in range(_RPW):
            xj = jnp.sum(jnp.where(lanes == 2 * j, pv, zero))
            yj = jnp.sum(jnp.where(lanes == 2 * j + 1, pv, zero))
            sels.append(xj < 128)
            xlos.append(jnp.minimum(xj, 127))
            phi = yj * 128 + jnp.maximum(xj - 96, 0)
            for h in range(2):
                for k in range(_HALF // 16):
                    c = h * _HALF + k * 16 + lanes
                    ridx_v[2 * j + h, pl.ds(k * 16, 16)] = (
                        (base + c) * _H + yj
                    )
            for ci in range(_CPR):
                k = j * _CPR + ci
                ihi_v[k // 8, pl.ds((k % 8) * 16, 16)] = (
                    (base + ci * 16 + lanes) * _PLANE + phi
                )
        hi_copies = [
            pltpu.async_copy(
                hi_hbm.at[ihi_v.at[t]], chi_v.at[pl.ds(t * 128, 128)], semh
            )
            for t in range(_NT)
        ]

        def fire(j, buf):
            return [
                pltpu.async_copy(
                    lo.at[ridx_v.at[2 * j + h]],
                    stage_v.at[buf, pl.ds(h * _HALF, _HALF)],
                    sems[buf],
                )
                for h in range(2)
            ]

        pending = fire(0, 0)
        for j in range(_RPW):
            nxt = fire(j + 1, (j + 1) % 2) if j + 1 < _RPW else None
            for cpy in pending:
                cpy.wait()
            buf = j % 2
            xvec = jnp.full((16,), xlos[j], jnp.int32)
            for ci in range(_CPR):
                vals = plsc.load_gather(
                    stage_v.at[buf], [ci * 16 + lanes, xvec]
                )
                out_v[pl.ds(j * _C + ci * 16, 16)] = vals
            pending = nxt
        for cpy in hi_copies:
            cpy.wait()
        for j in range(_RPW):
            sel = jnp.full((16,), sels[j], jnp.bool_)
            for ci in range(_CPR):
                o = j * _C + ci * 16
                out_v[pl.ds(o, 16)] = jnp.where(
                    sel, out_v[pl.ds(o, 16)], chi_v[pl.ds(o, 16)]
                )
        pltpu.sync_copy(out_v, out_hbm.at[pl.ds(wid * _NT * 128, _NT * 128)])

    return crop(imgs_rows, hi_flat, pts_flat)


def kernel(imgs, batch_points):
    imgs_rows = imgs.reshape(_B * _C * _H, _W)
    hi_flat = imgs[:, :, :, 96:].reshape(-1)
    pts_flat = batch_points.astype(jnp.int32).reshape(-1)
    out = _crop_call(imgs_rows, hi_flat, pts_flat)
    return out.reshape(_ROWS, _C), _ROWS
